# baseline (speedup 1.0000x reference)
import functools

import jax
import jax.numpy as jnp
from jax import lax
from jax.experimental import pallas as pl
from jax.experimental.pallas import tpu as pltpu
from jax.experimental.pallas import tpu_sc as plsc

_N = 10000
_E = 160000
_IN = 128
_OUT = 8
_DE = 16
_NG = 8

_NC = 2
_NS = 16
_L = 16
_NW = _NC * _NS
_CHUNK = 128
_NROWS = _E // _CHUNK
_BCH = _NROWS // _NW
_XTRA = _NROWS - _BCH * _NW
_MAXCH = _BCH + 1
_NBUF = 4
_ZC = 144
_ZPC = 80
_ACC_ROWS = 10112
_RPT = _ACC_ROWS // _NS
_MC = 16


def _mm_body(x_ref, w_ref, o_ref):
    o_ref[...] = jnp.dot(x_ref[...], w_ref[...],
                         preferred_element_type=jnp.float32)


def _c16(v):
    return jnp.full((_L,), v, jnp.int32)


def _sc_body(zr_hbm, ei_hbm, ea_hbm, out_hbm, *refs):
    src_v, dst_v = refs[0], refs[1]
    ea_s = refs[2:2 + _NBUF]
    zg_s = refs[2 + _NBUF:2 + 2 * _NBUF]
    msg_s = refs[2 + 2 * _NBUF:2 + 3 * _NBUF]
    stage_v = refs[2 + 3 * _NBUF]
    acc_sh = refs[3 + 3 * _NBUF]
    sem_e = refs[4 + 3 * _NBUF:4 + 3 * _NBUF + _NBUF]
    sem_g = refs[4 + 4 * _NBUF:4 + 5 * _NBUF]
    sem_s = refs[4 + 5 * _NBUF:4 + 6 * _NBUF]

    cid = lax.axis_index("c")
    sid = lax.axis_index("s")
    wid = sid * _NC + cid
    nch = jnp.where(wid < _XTRA, _BCH + 1, _BCH)
    r0 = wid * _BCH + jnp.minimum(wid, _XTRA)
    r0c = jnp.minimum(r0, _NROWS - _MAXCH)
    off = r0 - r0c

    zero16 = jnp.zeros((_L,), jnp.float32)
    ones16 = jnp.ones((_L,), jnp.float32)

    pltpu.sync_copy(ei_hbm.at[0, pl.ds(r0c, _MAXCH)], src_v)
    pltpu.sync_copy(ei_hbm.at[1, pl.ds(r0c, _MAXCH)], dst_v)

    def _z_acc(i, c):
        stage_v[i, :] = zero16
        return c
    lax.fori_loop(0, _RPT, _z_acc, 0)

    def _z_msg(i, c):
        for s in range(_NBUF):
            msg_s[s][i, :] = zero16
        return c
    lax.fori_loop(0, _CHUNK, _z_msg, 0)
    for bi in range(_CHUNK // _L):
        rows = lax.iota(jnp.int32, _L) + (bi * _L)
        for s in range(_NBUF):
            plsc.store_scatter(msg_s[s], [rows, _c16(8)], ones16)

    pltpu.sync_copy(stage_v, acc_sh.at[pl.ds(sid * _RPT, _RPT)])
    plsc.subcore_barrier()

    def _start(c, s):
        lc = jnp.minimum(c, nch - 1)
        pltpu.async_copy(
            ea_hbm.at[:, pl.ds((r0 + lc) * _CHUNK, _CHUNK)], ea_s[s],
            sem_e[s])
        pltpu.async_copy(zr_hbm.at[src_v.at[off + lc]], zg_s[s], sem_g[s])

    def _wait_in(s):
        pltpu.make_async_copy(
            ea_hbm.at[:, pl.ds(0, _CHUNK)], ea_s[s], sem_e[s]).wait()
        pltpu.make_async_copy(
            zr_hbm.at[src_v.at[0]], zg_s[s], sem_g[s]).wait()

    def _wait_sc(s):
        pltpu.make_async_copy(
            msg_s[s], acc_sh.at[dst_v.at[0]], sem_s[s]).wait()

    def _compute(s):
        @plsc.parallel_loop(0, _CHUNK // _L)
        def _block(bi):
            rows = lax.iota(jnp.int32, _L) + bi * _L
            eavs = [ea_s[s][d, pl.ds(bi * _L, _L)] for d in range(_DE)]
            himask = jnp.full((_L,), -65536, jnp.int32)
            for o2 in range(_OUT // 2):
                lo_a = hi_a = lo_b = hi_b = None
                for d in range(_DE + 1):
                    pi = plsc.load_gather(
                        zg_s[s], [rows, _c16(d * 4 + o2)])
                    fl = plsc.bitcast(pi << 16, jnp.float32)
                    fh = plsc.bitcast(pi & himask, jnp.float32)
                    if d < _DE:
                        fl = eavs[d] * fl
                        fh = eavs[d] * fh
                    if d < 9:
                        lo_a = fl if lo_a is None else lo_a + fl
                        hi_a = fh if hi_a is None else hi_a + fh
                    else:
                        lo_b = fl if lo_b is None else lo_b + fl
                        hi_b = fh if hi_b is None else hi_b + fh
                plsc.store_scatter(
                    msg_s[s], [rows, _c16(2 * o2)], lo_a + lo_b)
                plsc.store_scatter(
                    msg_s[s], [rows, _c16(2 * o2 + 1)], hi_a + hi_b)

    for s in range(_NBUF):
        _start(s, s)

    def _group(i, carry):
        for s in range(_NBUF):
            c = i * _NBUF + s
            _wait_in(s)

            @pl.when(i > 0)
            def _():
                _wait_sc(s)
            _compute(s)

            @pl.when(c < nch)
            def _():
                pltpu.async_copy(msg_s[s], acc_sh.at[dst_v.at[off + c]],
                                 sem_s[s], add=True)
            _start(c + _NBUF, s)
        return carry

    lax.fori_loop(0, _MAXCH // _NBUF, _group, 0)

    for s in range(_NBUF):
        _wait_in(s)
        last_c = _MAXCH - _NBUF + s

        @pl.when(last_c < nch)
        def _():
            _wait_sc(s)

    plsc.subcore_barrier()
    rr = sid * _RPT
    pltpu.sync_copy(acc_sh.at[pl.ds(rr, _RPT)], stage_v)
    pltpu.sync_copy(stage_v, out_hbm.at[cid, pl.ds(rr, _RPT)])


_sc_edges = functools.partial(
    pl.kernel,
    out_type=jax.ShapeDtypeStruct((_NC, _ACC_ROWS, _MC), jnp.float32),
    mesh=plsc.VectorSubcoreMesh(core_axis_name="c", subcore_axis_name="s"),
    scratch_types=(
        [
            pltpu.VMEM((_MAXCH, _CHUNK), jnp.int32),
            pltpu.VMEM((_MAXCH, _CHUNK), jnp.int32),
        ]
        + [pltpu.VMEM((_DE, _CHUNK), jnp.float32)] * _NBUF
        + [pltpu.VMEM((_CHUNK, _ZPC), jnp.int32)] * _NBUF
        + [pltpu.VMEM((_CHUNK, _MC), jnp.float32)] * _NBUF
        + [
            pltpu.VMEM((_RPT, _MC), jnp.float32),
            pltpu.VMEM_SHARED((_ACC_ROWS, _MC), jnp.float32),
        ]
        + [pltpu.SemaphoreType.DMA] * (3 * _NBUF)
    ),
    compiler_params=pltpu.CompilerParams(
        use_tc_tiling_on_sc=False, needs_layout_passes=False),
)(_sc_body)


def _fin_body(p_ref, zr_ref, bias_ref, batch_ref, o_ref):
    acc = p_ref[0] + p_ref[1]
    acc = acc[:_N]
    agg = acc[:, :_OUT] / jnp.maximum(acc[:, _OUT:_OUT + 1], 1.0)
    out = agg + zr_ref[:, 136:144] + bias_ref[...]
    b = batch_ref[...]
    gid = lax.broadcasted_iota(jnp.int32, (_NG, _N), 0)
    onehot = (b == gid).astype(jnp.float32)
    ps = jnp.dot(onehot, out, preferred_element_type=jnp.float32)
    gc = jnp.sum(onehot, axis=1, keepdims=True)
    o_ref[...] = ps / jnp.maximum(gc, 1.0)


def kernel(x, edge_index, edge_attr, batch, edge_mlp_W, edge_mlp_b, root,
           bias):
    W2 = edge_mlp_W.reshape(_DE, _IN, _OUT).transpose(1, 0, 2)
    W2 = W2.reshape(_IN, _DE * _OUT)
    Wcat = jnp.concatenate(
        [W2, edge_mlp_b.reshape(_IN, _OUT), root], axis=1)

    zr = pl.pallas_call(
        _mm_body,
        grid=(5,),
        in_specs=[
            pl.BlockSpec((2000, _IN), lambda i: (i, 0)),
            pl.BlockSpec((_IN, _ZC), lambda i: (0, 0)),
        ],
        out_specs=pl.BlockSpec((2000, _ZC), lambda i: (i, 0)),
        out_shape=jax.ShapeDtypeStruct((_N, _ZC), jnp.float32),
    )(x, Wcat)

    ei = edge_index.astype(jnp.int32).reshape(2, _NROWS, _CHUNK)
    ea = edge_attr.T

    zb = zr.astype(jnp.bfloat16)
    zb = jnp.concatenate(
        [zb, jnp.zeros((_N, 2 * _ZPC - _ZC), jnp.bfloat16)], axis=1)
    zp = lax.bitcast_convert_type(zb.reshape(_N, _ZPC, 2), jnp.int32)

    partials = _sc_edges(zp, ei, ea)

    out88 = pl.pallas_call(
        _fin_body,
        out_shape=jax.ShapeDtypeStruct((_NG, _OUT), jnp.float32),
    )(partials, zr, bias.reshape(1, _OUT),
      batch.astype(jnp.int32).reshape(1, _N))
    return out88.reshape(-1)


# speedup vs baseline: 1.1568x; 1.1568x over previous
import functools

import jax
import jax.numpy as jnp
from jax import lax
from jax.experimental import pallas as pl
from jax.experimental.pallas import tpu as pltpu
from jax.experimental.pallas import tpu_sc as plsc

_N = 10000
_E = 160000
_IN = 128
_OUT = 8
_DE = 16
_NG = 8

_NC = 2
_NS = 16
_L = 16
_NW = _NC * _NS
_CHUNK = 128
_NROWS = _E // _CHUNK
_BCH = _NROWS // _NW
_XTRA = _NROWS - _BCH * _NW
_MAXCH = _BCH + 1
_NBUF = 4
_ZC = 144
_ZPC = 80
_ACC_ROWS = 10112
_RPT = _ACC_ROWS // _NS
_MC = 16


def _mm_body(x_ref, w_ref, o_ref):
    o_ref[...] = jnp.dot(x_ref[...], w_ref[...],
                         preferred_element_type=jnp.float32)


def _c16(v):
    return jnp.full((_L,), v, jnp.int32)


def _sc_body(zr_hbm, ei_hbm, ea_hbm, out_hbm, *refs):
    src_v, dst_v = refs[0], refs[1]
    ea_s = refs[2:2 + _NBUF]
    zg_s = refs[2 + _NBUF:2 + 2 * _NBUF]
    msg_s = refs[2 + 2 * _NBUF:2 + 3 * _NBUF]
    stage_v = refs[2 + 3 * _NBUF]
    acc_sh = refs[3 + 3 * _NBUF]
    sem_e = refs[4 + 3 * _NBUF:4 + 3 * _NBUF + _NBUF]
    sem_g = refs[4 + 4 * _NBUF:4 + 5 * _NBUF]
    sem_s = refs[4 + 5 * _NBUF:4 + 6 * _NBUF]

    cid = lax.axis_index("c")
    sid = lax.axis_index("s")
    wid = sid * _NC + cid
    nch = jnp.where(wid < _XTRA, _BCH + 1, _BCH)
    r0 = wid * _BCH + jnp.minimum(wid, _XTRA)
    r0c = jnp.minimum(r0, _NROWS - _MAXCH)
    off = r0 - r0c

    zero16 = jnp.zeros((_L,), jnp.float32)
    ones16 = jnp.ones((_L,), jnp.float32)

    pltpu.sync_copy(ei_hbm.at[0, pl.ds(r0c, _MAXCH)], src_v)
    pltpu.sync_copy(ei_hbm.at[1, pl.ds(r0c, _MAXCH)], dst_v)

    def _z_acc(i, c):
        stage_v[i, :] = zero16
        return c
    lax.fori_loop(0, _RPT, _z_acc, 0)

    def _z_msg(i, c):
        for s in range(_NBUF):
            msg_s[s][i, :] = zero16
        return c
    lax.fori_loop(0, _CHUNK, _z_msg, 0)
    for bi in range(_CHUNK // _L):
        rows = lax.iota(jnp.int32, _L) + (bi * _L)
        for s in range(_NBUF):
            plsc.store_scatter(msg_s[s], [rows, _c16(8)], ones16)

    pltpu.sync_copy(stage_v, acc_sh.at[pl.ds(sid * _RPT, _RPT)])
    plsc.subcore_barrier()

    def _start(c, s):
        lc = jnp.minimum(c, nch - 1)
        pltpu.async_copy(
            ea_hbm.at[:, pl.ds((r0 + lc) * _CHUNK, _CHUNK)], ea_s[s],
            sem_e[s])
        pltpu.async_copy(zr_hbm.at[src_v.at[off + lc]], zg_s[s], sem_g[s])

    def _wait_in(s):
        pltpu.make_async_copy(
            ea_hbm.at[:, pl.ds(0, _CHUNK)], ea_s[s], sem_e[s]).wait()
        pltpu.make_async_copy(
            zr_hbm.at[src_v.at[0]], zg_s[s], sem_g[s]).wait()

    def _wait_sc(s):
        pltpu.make_async_copy(
            msg_s[s], acc_sh.at[dst_v.at[0]], sem_s[s]).wait()

    def _compute(s):
        @plsc.parallel_loop(0, _CHUNK // _L)
        def _block(bi):
            rows = lax.iota(jnp.int32, _L) + bi * _L
            eavs = [ea_s[s][d, pl.ds(bi * _L, _L)] for d in range(_DE)]
            for o in range(_OUT):
                terms = [plsc.load_gather(zg_s[s], [rows, _c16(128 + o)])]
                terms += [
                    eavs[d] * plsc.load_gather(
                        zg_s[s], [rows, _c16(d * 8 + o)])
                    for d in range(_DE)
                ]
                while len(terms) > 1:
                    nxt = [terms[k] + terms[k + 1]
                           for k in range(0, len(terms) - 1, 2)]
                    if len(terms) % 2:
                        nxt.append(terms[-1])
                    terms = nxt
                plsc.store_scatter(msg_s[s], [rows, _c16(o)], terms[0])

    for s in range(_NBUF):
        _start(s, s)

    def _group(i, carry):
        for s in range(_NBUF):
            c = i * _NBUF + s
            _wait_in(s)

            @pl.when(i > 0)
            def _():
                _wait_sc(s)
            _compute(s)

            @pl.when(c < nch)
            def _():
                pltpu.async_copy(msg_s[s], acc_sh.at[dst_v.at[off + c]],
                                 sem_s[s], add=True)
            _start(c + _NBUF, s)
        return carry

    lax.fori_loop(0, _MAXCH // _NBUF, _group, 0)

    for s in range(_NBUF):
        _wait_in(s)
        last_c = _MAXCH - _NBUF + s

        @pl.when(last_c < nch)
        def _():
            _wait_sc(s)

    plsc.subcore_barrier()
    rr = sid * _RPT
    pltpu.sync_copy(acc_sh.at[pl.ds(rr, _RPT)], stage_v)
    pltpu.sync_copy(stage_v, out_hbm.at[cid, pl.ds(rr, _RPT)])


_sc_edges = functools.partial(
    pl.kernel,
    out_type=jax.ShapeDtypeStruct((_NC, _ACC_ROWS, _MC), jnp.float32),
    mesh=plsc.VectorSubcoreMesh(core_axis_name="c", subcore_axis_name="s"),
    scratch_types=(
        [
            pltpu.VMEM((_MAXCH, _CHUNK), jnp.int32),
            pltpu.VMEM((_MAXCH, _CHUNK), jnp.int32),
        ]
        + [pltpu.VMEM((_DE, _CHUNK), jnp.float32)] * _NBUF
        + [pltpu.VMEM((_CHUNK, _ZC), jnp.float32)] * _NBUF
        + [pltpu.VMEM((_CHUNK, _MC), jnp.float32)] * _NBUF
        + [
            pltpu.VMEM((_RPT, _MC), jnp.float32),
            pltpu.VMEM_SHARED((_ACC_ROWS, _MC), jnp.float32),
        ]
        + [pltpu.SemaphoreType.DMA] * (3 * _NBUF)
    ),
    compiler_params=pltpu.CompilerParams(
        use_tc_tiling_on_sc=False, needs_layout_passes=False),
)(_sc_body)


def _fin_body(p_ref, zr_ref, bias_ref, batch_ref, o_ref):
    acc = p_ref[0] + p_ref[1]
    acc = acc[:_N]
    agg = acc[:, :_OUT] / jnp.maximum(acc[:, _OUT:_OUT + 1], 1.0)
    out = agg + zr_ref[:, 136:144] + bias_ref[...]
    b = batch_ref[...]
    gid = lax.broadcasted_iota(jnp.int32, (_NG, _N), 0)
    onehot = (b == gid).astype(jnp.float32)
    ps = jnp.dot(onehot, out, preferred_element_type=jnp.float32)
    gc = jnp.sum(onehot, axis=1, keepdims=True)
    o_ref[...] = ps / jnp.maximum(gc, 1.0)


def kernel(x, edge_index, edge_attr, batch, edge_mlp_W, edge_mlp_b, root,
           bias):
    W2 = edge_mlp_W.reshape(_DE, _IN, _OUT).transpose(1, 0, 2)
    W2 = W2.reshape(_IN, _DE * _OUT)
    Wcat = jnp.concatenate(
        [W2, edge_mlp_b.reshape(_IN, _OUT), root], axis=1)

    zr = pl.pallas_call(
        _mm_body,
        out_shape=jax.ShapeDtypeStruct((_N, _ZC), jnp.float32),
    )(x, Wcat)

    ei = edge_index.astype(jnp.int32).reshape(2, _NROWS, _CHUNK)
    ea = edge_attr.T

    partials = _sc_edges(zr, ei, ea)

    out88 = pl.pallas_call(
        _fin_body,
        out_shape=jax.ShapeDtypeStruct((_NG, _OUT), jnp.float32),
    )(partials, zr, bias.reshape(1, _OUT),
      batch.astype(jnp.int32).reshape(1, _N))
    return out88.reshape(-1)


# speedup vs baseline: 1.1585x; 1.0015x over previous
import functools

import jax
import jax.numpy as jnp
from jax import lax
from jax.experimental import pallas as pl
from jax.experimental.pallas import tpu as pltpu
from jax.experimental.pallas import tpu_sc as plsc

_N = 10000
_E = 160000
_IN = 128
_OUT = 8
_DE = 16
_NG = 8

_NC = 2
_NS = 16
_L = 16
_NW = _NC * _NS
_CHUNK = 128
_NROWS = _E // _CHUNK
_BCH = _NROWS // _NW
_XTRA = _NROWS - _BCH * _NW
_MAXCH = _BCH + 1
_NBUF = 4
_ZC = 144
_ACC_ROWS = 10112
_RPT = _ACC_ROWS // _NS
_MC = 16


def _mm_body(x_ref, w_ref, o_ref):
    o_ref[...] = jnp.dot(x_ref[...], w_ref[...],
                         preferred_element_type=jnp.float32)


def _c16(v):
    return jnp.full((_L,), v, jnp.int32)


def _sc_body(zr_hbm, ei_hbm, ea_hbm, out_hbm, *refs):
    src_v, dst_v = refs[0], refs[1]
    ea_s = refs[2:2 + _NBUF]
    zg_s = refs[2 + _NBUF:2 + 2 * _NBUF]
    msg_s = refs[2 + 2 * _NBUF:2 + 3 * _NBUF]
    stage_v = refs[2 + 3 * _NBUF]
    acc_sh = refs[3 + 3 * _NBUF]
    sem_e = refs[4 + 3 * _NBUF:4 + 3 * _NBUF + _NBUF]
    sem_g = refs[4 + 4 * _NBUF:4 + 5 * _NBUF]
    sem_s = refs[4 + 5 * _NBUF:4 + 6 * _NBUF]

    cid = lax.axis_index("c")
    sid = lax.axis_index("s")
    wid = sid * _NC + cid
    nch = jnp.where(wid < _XTRA, _BCH + 1, _BCH)
    r0 = wid * _BCH + jnp.minimum(wid, _XTRA)
    r0c = jnp.minimum(r0, _NROWS - _MAXCH)
    off = r0 - r0c

    zero16 = jnp.zeros((_L,), jnp.float32)
    ones16 = jnp.ones((_L,), jnp.float32)

    pltpu.sync_copy(ei_hbm.at[0, pl.ds(r0c, _MAXCH)], src_v)
    pltpu.sync_copy(ei_hbm.at[1, pl.ds(r0c, _MAXCH)], dst_v)

    def _z_acc(i, c):
        stage_v[i, :] = zero16
        return c
    lax.fori_loop(0, _RPT, _z_acc, 0)

    def _z_msg(i, c):
        for s in range(_NBUF):
            msg_s[s][i, :] = zero16
        return c
    lax.fori_loop(0, _CHUNK, _z_msg, 0)
    for bi in range(_CHUNK // _L):
        rows = lax.iota(jnp.int32, _L) + (bi * _L)
        for s in range(_NBUF):
            plsc.store_scatter(msg_s[s], [rows, _c16(8)], ones16)

    pltpu.sync_copy(stage_v, acc_sh.at[pl.ds(sid * _RPT, _RPT)])
    plsc.subcore_barrier()

    def _start(c, s):
        lc = jnp.minimum(c, nch - 1)
        pltpu.async_copy(
            ea_hbm.at[:, pl.ds((r0 + lc) * _CHUNK, _CHUNK)], ea_s[s],
            sem_e[s])
        pltpu.async_copy(zr_hbm.at[src_v.at[off + lc]], zg_s[s], sem_g[s])

    def _wait_in(s):
        pltpu.make_async_copy(
            ea_hbm.at[:, pl.ds(0, _CHUNK)], ea_s[s], sem_e[s]).wait()
        pltpu.make_async_copy(
            zr_hbm.at[src_v.at[0]], zg_s[s], sem_g[s]).wait()

    def _wait_sc(s):
        pltpu.make_async_copy(
            msg_s[s], acc_sh.at[dst_v.at[0]], sem_s[s]).wait()

    def _compute(s):
        @plsc.parallel_loop(0, _CHUNK // _L)
        def _block(bi):
            rows = lax.iota(jnp.int32, _L) + bi * _L
            eavs = [ea_s[s][d, pl.ds(bi * _L, _L)] for d in range(_DE)]
            for o in range(_OUT):
                terms = [plsc.load_gather(zg_s[s], [rows, _c16(128 + o)])]
                terms += [
                    eavs[d] * plsc.load_gather(
                        zg_s[s], [rows, _c16(d * 8 + o)])
                    for d in range(_DE)
                ]
                while len(terms) > 1:
                    nxt = [terms[k] + terms[k + 1]
                           for k in range(0, len(terms) - 1, 2)]
                    if len(terms) % 2:
                        nxt.append(terms[-1])
                    terms = nxt
                plsc.store_scatter(msg_s[s], [rows, _c16(o)], terms[0])

    for s in range(_NBUF):
        _start(s, s)

    def _group(i, carry):
        for s in range(_NBUF):
            c = i * _NBUF + s
            _wait_in(s)

            @pl.when(i > 0)
            def _():
                _wait_sc(s)
            _compute(s)

            @pl.when(c < nch)
            def _():
                pltpu.async_copy(msg_s[s], acc_sh.at[dst_v.at[off + c]],
                                 sem_s[s], add=True)
            _start(c + _NBUF, s)
        return carry

    lax.fori_loop(0, _MAXCH // _NBUF, _group, 0)

    for s in range(_NBUF):
        _wait_in(s)
        last_c = _MAXCH - _NBUF + s

        @pl.when(last_c < nch)
        def _():
            _wait_sc(s)

    plsc.subcore_barrier()
    rr = sid * _RPT
    pltpu.sync_copy(acc_sh.at[pl.ds(rr, _RPT)], stage_v)
    pltpu.sync_copy(stage_v, out_hbm.at[cid, pl.ds(rr, _RPT)])


_sc_edges = functools.partial(
    pl.kernel,
    out_type=jax.ShapeDtypeStruct((_NC, _ACC_ROWS, _MC), jnp.float32),
    mesh=plsc.VectorSubcoreMesh(core_axis_name="c", subcore_axis_name="s"),
    scratch_types=(
        [
            pltpu.VMEM((_MAXCH, _CHUNK), jnp.int32),
            pltpu.VMEM((_MAXCH, _CHUNK), jnp.int32),
        ]
        + [pltpu.VMEM((_DE, _CHUNK), jnp.float32)] * _NBUF
        + [pltpu.VMEM((_CHUNK, _ZC), jnp.float32)] * _NBUF
        + [pltpu.VMEM((_CHUNK, _MC), jnp.float32)] * _NBUF
        + [
            pltpu.VMEM((_RPT, _MC), jnp.float32),
            pltpu.VMEM_SHARED((_ACC_ROWS, _MC), jnp.float32),
        ]
        + [pltpu.SemaphoreType.DMA] * (3 * _NBUF)
    ),
    compiler_params=pltpu.CompilerParams(
        use_tc_tiling_on_sc=False, needs_layout_passes=False),
)(_sc_body)


def _fin_body(p_ref, zr_ref, bias_ref, batch_ref, o_ref):
    acc = p_ref[0] + p_ref[1]
    acc = acc[:_N]
    agg = acc[:, :_OUT] / jnp.maximum(acc[:, _OUT:_OUT + 1], 1.0)
    out = agg + zr_ref[:, 136:144] + bias_ref[...]
    b = batch_ref[...]
    gid = lax.broadcasted_iota(jnp.int32, (_NG, _N), 0)
    onehot = (b == gid).astype(jnp.float32)
    ps = jnp.dot(onehot, out, preferred_element_type=jnp.float32)
    gc = jnp.sum(onehot, axis=1, keepdims=True)
    o_ref[...] = ps / jnp.maximum(gc, 1.0)


def kernel(x, edge_index, edge_attr, batch, edge_mlp_W, edge_mlp_b, root,
           bias):
    W2 = edge_mlp_W.reshape(_DE, _IN, _OUT).transpose(1, 0, 2)
    W2 = W2.reshape(_IN, _DE * _OUT)
    Wcat = jnp.concatenate(
        [W2, edge_mlp_b.reshape(_IN, _OUT), root], axis=1)

    zr = pl.pallas_call(
        _mm_body,
        out_shape=jax.ShapeDtypeStruct((_N, _ZC), jnp.float32),
    )(x, Wcat)

    ei = edge_index.astype(jnp.int32).reshape(2, _NROWS, _CHUNK)
    ea = edge_attr.T

    partials = _sc_edges(zr, ei, ea)

    out88 = pl.pallas_call(
        _fin_body,
        out_shape=jax.ShapeDtypeStruct((_NG, _OUT), jnp.float32),
    )(partials, zr, bias.reshape(1, _OUT),
      batch.astype(jnp.int32).reshape(1, _N))
    return out88.reshape(-1)


# speedup vs baseline: 1.1832x; 1.0213x over previous
import functools

import jax
import jax.numpy as jnp
from jax import lax
from jax.experimental import pallas as pl
from jax.experimental.pallas import tpu as pltpu
from jax.experimental.pallas import tpu_sc as plsc

_N = 10000
_E = 160000
_IN = 128
_OUT = 8
_DE = 16
_NG = 8

_NC = 2
_NS = 16
_L = 16
_NW = _NC * _NS
_CHUNK = 128
_NROWS = _E // _CHUNK
_BCH = _NROWS // _NW
_XTRA = _NROWS - _BCH * _NW
_MAXCH = _BCH + 1
_NBUF = 4
_ZC = 144
_ACC_ROWS = 10112
_RPT = _ACC_ROWS // _NS
_MC = 16


def _mm_body(x_ref, w_ref, o_ref):
    o_ref[...] = jnp.dot(x_ref[...], w_ref[...],
                         preferred_element_type=jnp.float32)


def _c16(v):
    return jnp.full((_L,), v, jnp.int32)


def _sc_body(zr_hbm, ei_hbm, ea_hbm, out_hbm, *refs):
    src_v, dst_v = refs[0], refs[1]
    ea_s = refs[2:2 + _NBUF]
    zg_s = refs[2 + _NBUF:2 + 2 * _NBUF]
    msg_s = refs[2 + 2 * _NBUF:2 + 3 * _NBUF]
    stage_v = refs[2 + 3 * _NBUF]
    acc_sh = refs[3 + 3 * _NBUF]
    sem_e = refs[4 + 3 * _NBUF:4 + 3 * _NBUF + _NBUF]
    sem_g = refs[4 + 4 * _NBUF:4 + 5 * _NBUF]
    sem_s = refs[4 + 5 * _NBUF:4 + 6 * _NBUF]

    cid = lax.axis_index("c")
    sid = lax.axis_index("s")
    wid = sid * _NC + cid
    nch = jnp.where(wid < _XTRA, _BCH + 1, _BCH)
    r0 = wid * _BCH + jnp.minimum(wid, _XTRA)
    r0c = jnp.minimum(r0, _NROWS - _MAXCH)
    off = r0 - r0c

    zero16 = jnp.zeros((_L,), jnp.float32)
    ones16 = jnp.ones((_L,), jnp.float32)

    pltpu.sync_copy(ei_hbm.at[0, pl.ds(r0c, _MAXCH)], src_v)
    pltpu.sync_copy(ei_hbm.at[1, pl.ds(r0c, _MAXCH)], dst_v)

    def _start(c, s):
        lc = jnp.minimum(c, nch - 1)
        pltpu.async_copy(
            ea_hbm.at[:, pl.ds((r0 + lc) * _CHUNK, _CHUNK)], ea_s[s],
            sem_e[s])
        pltpu.async_copy(zr_hbm.at[src_v.at[off + lc]], zg_s[s], sem_g[s])

    def _wait_in(s):
        pltpu.make_async_copy(
            ea_hbm.at[:, pl.ds(0, _CHUNK)], ea_s[s], sem_e[s]).wait()
        pltpu.make_async_copy(
            zr_hbm.at[src_v.at[0]], zg_s[s], sem_g[s]).wait()

    def _wait_sc(s):
        pltpu.make_async_copy(
            msg_s[s], acc_sh.at[dst_v.at[0]], sem_s[s]).wait()

    def _compute(s):
        @plsc.parallel_loop(0, _CHUNK // _L)
        def _block(bi):
            rows = lax.iota(jnp.int32, _L) + bi * _L
            eavs = [ea_s[s][d, pl.ds(bi * _L, _L)] for d in range(_DE)]
            for o in range(_OUT):
                terms = [plsc.load_gather(zg_s[s], [rows, _c16(128 + o)])]
                terms += [
                    eavs[d] * plsc.load_gather(
                        zg_s[s], [rows, _c16(d * 8 + o)])
                    for d in range(_DE)
                ]
                while len(terms) > 1:
                    nxt = [terms[k] + terms[k + 1]
                           for k in range(0, len(terms) - 1, 2)]
                    if len(terms) % 2:
                        nxt.append(terms[-1])
                    terms = nxt
                plsc.store_scatter(msg_s[s], [rows, _c16(o)], terms[0])

    for s in range(_NBUF):
        _start(s, s)

    def _z_acc(i, c):
        stage_v[i, :] = zero16
        return c
    lax.fori_loop(0, _RPT, _z_acc, 0)

    def _z_msg(i, c):
        for s in range(_NBUF):
            msg_s[s][i, :] = zero16
        return c
    lax.fori_loop(0, _CHUNK, _z_msg, 0)
    for bi in range(_CHUNK // _L):
        rows = lax.iota(jnp.int32, _L) + (bi * _L)
        for s in range(_NBUF):
            plsc.store_scatter(msg_s[s], [rows, _c16(8)], ones16)

    pltpu.sync_copy(stage_v, acc_sh.at[pl.ds(sid * _RPT, _RPT)])
    plsc.subcore_barrier()

    def _group(i, carry):
        for s in range(_NBUF):
            c = i * _NBUF + s
            _wait_in(s)

            @pl.when(i > 0)
            def _():
                _wait_sc(s)
            _compute(s)

            @pl.when(c < nch)
            def _():
                pltpu.async_copy(msg_s[s], acc_sh.at[dst_v.at[off + c]],
                                 sem_s[s], add=True)
            _start(c + _NBUF, s)
        return carry

    lax.fori_loop(0, _MAXCH // _NBUF, _group, 0)

    for s in range(_NBUF):
        _wait_in(s)
        last_c = _MAXCH - _NBUF + s

        @pl.when(last_c < nch)
        def _():
            _wait_sc(s)

    plsc.subcore_barrier()
    rr = sid * _RPT
    pltpu.sync_copy(acc_sh.at[pl.ds(rr, _RPT)], stage_v)
    pltpu.sync_copy(stage_v, out_hbm.at[cid, pl.ds(rr, _RPT)])


_sc_edges = functools.partial(
    pl.kernel,
    out_type=jax.ShapeDtypeStruct((_NC, _ACC_ROWS, _MC), jnp.float32),
    mesh=plsc.VectorSubcoreMesh(core_axis_name="c", subcore_axis_name="s"),
    scratch_types=(
        [
            pltpu.VMEM((_MAXCH, _CHUNK), jnp.int32),
            pltpu.VMEM((_MAXCH, _CHUNK), jnp.int32),
        ]
        + [pltpu.VMEM((_DE, _CHUNK), jnp.float32)] * _NBUF
        + [pltpu.VMEM((_CHUNK, _ZC), jnp.float32)] * _NBUF
        + [pltpu.VMEM((_CHUNK, _MC), jnp.float32)] * _NBUF
        + [
            pltpu.VMEM((_RPT, _MC), jnp.float32),
            pltpu.VMEM_SHARED((_ACC_ROWS, _MC), jnp.float32),
        ]
        + [pltpu.SemaphoreType.DMA] * (3 * _NBUF)
    ),
    compiler_params=pltpu.CompilerParams(
        use_tc_tiling_on_sc=False, needs_layout_passes=False),
)(_sc_body)


def _fin_body(p_ref, zr_ref, bias_ref, batch_ref, o_ref):
    acc = p_ref[0] + p_ref[1]
    acc = acc[:_N]
    agg = acc[:, :_OUT] / jnp.maximum(acc[:, _OUT:_OUT + 1], 1.0)
    out = agg + zr_ref[:, 136:144] + bias_ref[...]
    b = batch_ref[...]
    gid = lax.broadcasted_iota(jnp.int32, (_NG, _N), 0)
    onehot = (b == gid).astype(jnp.float32)
    ps = jnp.dot(onehot, out, preferred_element_type=jnp.float32)
    gc = jnp.sum(onehot, axis=1, keepdims=True)
    o_ref[...] = ps / jnp.maximum(gc, 1.0)


def kernel(x, edge_index, edge_attr, batch, edge_mlp_W, edge_mlp_b, root,
           bias):
    W2 = edge_mlp_W.reshape(_DE, _IN, _OUT).transpose(1, 0, 2)
    W2 = W2.reshape(_IN, _DE * _OUT)
    Wcat = jnp.concatenate(
        [W2, edge_mlp_b.reshape(_IN, _OUT), root], axis=1)

    zr = pl.pallas_call(
        _mm_body,
        out_shape=jax.ShapeDtypeStruct((_N, _ZC), jnp.float32),
    )(x, Wcat)

    ei = edge_index.astype(jnp.int32).reshape(2, _NROWS, _CHUNK)
    ea = edge_attr.T

    partials = _sc_edges(zr, ei, ea)

    out88 = pl.pallas_call(
        _fin_body,
        out_shape=jax.ShapeDtypeStruct((_NG, _OUT), jnp.float32),
    )(partials, zr, bias.reshape(1, _OUT),
      batch.astype(jnp.int32).reshape(1, _N))
    return out88.reshape(-1)
